# baseline (speedup 1.0000x reference)
import jax
import jax.numpy as jnp
from jax.experimental import pallas as pl
from jax.experimental.pallas import tpu as pltpu


def _shift_body(data_ref, x_ref, out_ref, sem0, sem1):
    n = data_ref.shape[0]
    bulk = pltpu.make_async_copy(
        data_ref.at[pl.ds(0, n - 128)],
        out_ref.at[pl.ds(128, n - 128)],
        sem0,
    )
    head = pltpu.make_async_copy(x_ref, out_ref.at[pl.ds(0, 128)], sem1)
    bulk.start()
    head.start()
    head.wait()
    bulk.wait()


def kernel(data, x):
    n, d = data.shape
    flat = pl.pallas_call(
        _shift_body,
        in_specs=[
            pl.BlockSpec(memory_space=pl.ANY),
            pl.BlockSpec(memory_space=pl.ANY),
        ],
        out_specs=pl.BlockSpec(memory_space=pl.ANY),
        out_shape=jax.ShapeDtypeStruct((n * d,), data.dtype),
        scratch_shapes=[pltpu.SemaphoreType.DMA, pltpu.SemaphoreType.DMA],
    )(data.reshape(-1), x)
    return flat.reshape(n, d)


# speedup vs baseline: 35.7951x; 35.7951x over previous
import jax
import jax.numpy as jnp
from jax.experimental import pallas as pl
from jax.experimental.pallas import tpu as pltpu

_C = 1 << 20
_NBUF = 4


def _shift_body(data_ref, x_ref, out_ref, bufs, lsems, ssems, hsem):
    total = data_ref.shape[0] - 128
    nc = (total + _C - 1) // _C

    def load(k):
        off = k * _C
        sz = min(_C, total - off)
        b = k % _NBUF
        return pltpu.make_async_copy(
            data_ref.at[pl.ds(off, sz)],
            bufs.at[b, pl.ds(0, sz)],
            lsems.at[b],
        )

    def store(k):
        off = k * _C
        sz = min(_C, total - off)
        b = k % _NBUF
        return pltpu.make_async_copy(
            bufs.at[b, pl.ds(0, sz)],
            out_ref.at[pl.ds(128 + off, sz)],
            ssems.at[b],
        )

    loads = [load(k) for k in range(nc)]
    stores = [store(k) for k in range(nc)]

    head = pltpu.make_async_copy(x_ref, out_ref.at[pl.ds(0, 128)], hsem)
    head.start()

    for k in range(min(_NBUF, nc)):
        loads[k].start()
    for k in range(nc):
        loads[k].wait()
        stores[k].start()
        nl = k + 1
        if _NBUF <= nl < nc:
            stores[nl - _NBUF].wait()
            loads[nl].start()
    for k in range(max(0, nc - _NBUF), nc):
        stores[k].wait()
    head.wait()


def kernel(data, x):
    n, d = data.shape
    flat = pl.pallas_call(
        _shift_body,
        in_specs=[
            pl.BlockSpec(memory_space=pl.ANY),
            pl.BlockSpec(memory_space=pl.ANY),
        ],
        out_specs=pl.BlockSpec(memory_space=pl.ANY),
        out_shape=jax.ShapeDtypeStruct((n * d,), data.dtype),
        scratch_shapes=[
            pltpu.VMEM((_NBUF, _C), jnp.float32),
            pltpu.SemaphoreType.DMA((_NBUF,)),
            pltpu.SemaphoreType.DMA((_NBUF,)),
            pltpu.SemaphoreType.DMA,
        ],
    )(data.reshape(-1), x)
    return flat.reshape(n, d)
